# baseline (speedup 1.0000x reference)
import functools

import jax
import jax.numpy as jnp
from jax import lax
from jax.experimental import pallas as pl
from jax.experimental.pallas import tpu as pltpu
from jax.experimental.pallas import tpu_sc as plsc

N = 10000
E = 320000
D = 128
DH = D // 2

NP = 10240
NC = 2
NS = 16
NW = NC * NS
K = 128
EP = 327680
NB_DEG = EP // NW // K
NB = EP // NW // K
CH = 20
NCH = NB // CH
RPT = NP // NS
DUMP = N

_mesh = plsc.VectorSubcoreMesh(core_axis_name="c", subcore_axis_name="s")


@functools.partial(
    pl.kernel,
    out_type=jax.ShapeDtypeStruct((NC, NP), jnp.float32),
    mesh=_mesh,
    scratch_types=[
        pltpu.VMEM((NB_DEG, K), jnp.int32),
        pltpu.VMEM((K,), jnp.float32),
        pltpu.VMEM_SHARED((NP,), jnp.float32),
        pltpu.SemaphoreType.DMA,
    ],
)
def _degree_kernel(col_hbm, zeros1_hbm, ones_hbm, dpart_hbm,
                   col_v, ones_v, acc_d, sem):
    cid = lax.axis_index("c")
    sid = lax.axis_index("s")
    wid = sid * NC + cid
    pltpu.sync_copy(zeros1_hbm, acc_d.at[pl.ds(sid * RPT, RPT)])
    pltpu.sync_copy(ones_hbm, ones_v)
    pltpu.sync_copy(col_hbm.at[wid], col_v)
    plsc.subcore_barrier()

    def body(j, _):
        pltpu.sync_copy(ones_v, acc_d.at[col_v.at[j]], add=True)
        return ()

    lax.fori_loop(0, NB_DEG, body, ())
    plsc.subcore_barrier()
    pltpu.sync_copy(acc_d.at[pl.ds(sid * RPT, RPT)],
                    dpart_hbm.at[cid, pl.ds(sid * RPT, RPT)])


@functools.partial(
    pl.kernel,
    out_type=jax.ShapeDtypeStruct((NC, NP, D), jnp.float32),
    mesh=_mesh,
    scratch_types=[
        pltpu.VMEM((CH + 1, K), jnp.int32),
        pltpu.VMEM((CH, K), jnp.int32),
        pltpu.VMEM((K, D), jnp.float32),
        pltpu.VMEM((K, D), jnp.float32),
        pltpu.VMEM_SHARED((NP, D), jnp.float32),
        pltpu.SemaphoreType.DMA,
        pltpu.SemaphoreType.DMA,
    ],
)
def _aggregate_kernel(y_hbm, row_hbm, col_hbm, zeros2_hbm, zeroi_hbm,
                      out_hbm, row_v, col_v, buf0, buf1, acc, sem0, sem1):
    cid = lax.axis_index("c")
    sid = lax.axis_index("s")
    wid = sid * NC + cid
    pltpu.sync_copy(zeros2_hbm, acc.at[pl.ds(sid * RPT, RPT)])
    pltpu.sync_copy(zeroi_hbm, row_v.at[CH])
    plsc.subcore_barrier()

    def chunk(c, _):
        pltpu.sync_copy(row_hbm.at[wid, c], row_v.at[pl.ds(0, CH)])
        pltpu.sync_copy(col_hbm.at[wid, c], col_v)
        pltpu.async_copy(y_hbm.at[row_v.at[0]], buf0, sem0)

        def body(j, _):
            b0 = 2 * j
            pltpu.async_copy(y_hbm.at[row_v.at[b0 + 1]], buf1, sem1)
            pltpu.make_async_copy(y_hbm.at[row_v.at[b0]], buf0, sem0).wait()
            pltpu.sync_copy(buf0, acc.at[col_v.at[b0]], add=True)
            pltpu.async_copy(y_hbm.at[row_v.at[b0 + 2]], buf0, sem0)
            pltpu.make_async_copy(y_hbm.at[row_v.at[b0 + 1]], buf1, sem1).wait()
            pltpu.sync_copy(buf1, acc.at[col_v.at[b0 + 1]], add=True)
            return ()

        lax.fori_loop(0, CH // 2, body, ())
        pltpu.make_async_copy(y_hbm.at[row_v.at[CH]], buf0, sem0).wait()
        return ()

    lax.fori_loop(0, NCH, chunk, ())
    plsc.subcore_barrier()
    pltpu.sync_copy(acc.at[pl.ds(sid * RPT, RPT)],
                    out_hbm.at[cid, pl.ds(sid * RPT, RPT)])


def _scale_matmul_body(d0_r, d1_r, x_r, w_r, y_r, dn_r):
    d = d0_r[...] + d1_r[...]
    dn = jnp.where(d > 0.0, lax.rsqrt(d), 0.0)
    xs = x_r[...] * dn[:, None]
    y_r[...] = lax.dot_general(xs, w_r[...], (((1,), (1,)), ((), ())),
                               preferred_element_type=jnp.float32)
    dn_r[...] = dn


BN = 512
_scale_matmul = pl.pallas_call(
    _scale_matmul_body,
    grid=(NP // BN,),
    in_specs=[
        pl.BlockSpec((BN,), lambda i: (i,)),
        pl.BlockSpec((BN,), lambda i: (i,)),
        pl.BlockSpec((BN, D), lambda i: (i, 0)),
        pl.BlockSpec((D, D), lambda i: (0, 0)),
    ],
    out_specs=[
        pl.BlockSpec((BN, D), lambda i: (i, 0)),
        pl.BlockSpec((BN,), lambda i: (i,)),
    ],
    out_shape=[
        jax.ShapeDtypeStruct((NP, D), jnp.float32),
        jax.ShapeDtypeStruct((NP,), jnp.float32),
    ],
)


def _finish_body(dn_r, p0_r, p1_r, b_r, out_r):
    agg = p0_r[...] + p1_r[...]
    out_r[...] = dn_r[...][:, None] * agg + b_r[...][None, :]


_finish = pl.pallas_call(
    _finish_body,
    grid=(NP // BN,),
    in_specs=[
        pl.BlockSpec((BN,), lambda i: (i,)),
        pl.BlockSpec((BN, D), lambda i: (i, 0)),
        pl.BlockSpec((BN, D), lambda i: (i, 0)),
        pl.BlockSpec((D,), lambda i: (0,)),
    ],
    out_specs=pl.BlockSpec((BN, D), lambda i: (i, 0)),
    out_shape=jax.ShapeDtypeStruct((NP, D), jnp.float32),
)


@jax.jit
def kernel(x, edge_index, x0, W, b):
    row = edge_index[0]
    col = edge_index[1]
    pad = EP - E
    row_p = jnp.concatenate([row, jnp.zeros((pad,), jnp.int32)])
    col_p = jnp.concatenate([col, jnp.full((pad,), DUMP, jnp.int32)])
    col3w = col_p.reshape(NW, NB_DEG, K)
    row4 = row_p.reshape(NW, NCH, CH, K)
    col4 = col_p.reshape(NW, NCH, CH, K)
    x_p = jnp.pad(x, ((0, NP - N), (0, 0)))

    zeros1 = jnp.zeros((RPT,), jnp.float32)
    zeros2 = jnp.zeros((RPT, D), jnp.float32)
    zeroi = jnp.zeros((K,), jnp.int32)
    ones = jnp.ones((K,), jnp.float32)

    d_part = _degree_kernel(col3w, zeros1, ones)
    y, dn = _scale_matmul(d_part[0], d_part[1], x_p, W)
    parts = _aggregate_kernel(y, row4, col4, zeros2, zeroi)
    out = _finish(dn, parts[0], parts[1], b)
    return out[:N]


# speedup vs baseline: 1.7108x; 1.7108x over previous
import functools

import jax
import jax.numpy as jnp
from jax import lax
from jax.experimental import pallas as pl
from jax.experimental.pallas import tpu as pltpu
from jax.experimental.pallas import tpu_sc as plsc

N = 10000
E = 320000
D = 128
DH = D // 2

NP = 10240
NC = 2
NS = 16
NW = NC * NS
K = 128
EP = 327680
NB_DEG = EP // NW // K
NB = EP // NW // K
CH = 40
NCH = NB // CH
RPT = NP // NS
DUMP = N

_mesh = plsc.VectorSubcoreMesh(core_axis_name="c", subcore_axis_name="s")


@functools.partial(
    pl.kernel,
    out_type=jax.ShapeDtypeStruct((NC, NP), jnp.float32),
    mesh=_mesh,
    scratch_types=[
        pltpu.VMEM((NB_DEG, K), jnp.int32),
        pltpu.VMEM((K,), jnp.float32),
        pltpu.VMEM_SHARED((NP,), jnp.float32),
        pltpu.SemaphoreType.DMA,
    ],
)
def _degree_kernel(col_hbm, zeros1_hbm, ones_hbm, dpart_hbm,
                   col_v, ones_v, acc_d, sem):
    cid = lax.axis_index("c")
    sid = lax.axis_index("s")
    wid = sid * NC + cid
    pltpu.sync_copy(zeros1_hbm, acc_d.at[pl.ds(sid * RPT, RPT)])
    pltpu.sync_copy(ones_hbm, ones_v)
    pltpu.sync_copy(col_hbm.at[wid], col_v)
    plsc.subcore_barrier()

    def body(j, _):
        pltpu.sync_copy(ones_v, acc_d.at[col_v.at[j]], add=True)
        return ()

    lax.fori_loop(0, NB_DEG, body, ())
    plsc.subcore_barrier()
    pltpu.sync_copy(acc_d.at[pl.ds(sid * RPT, RPT)],
                    dpart_hbm.at[cid, pl.ds(sid * RPT, RPT)])


@functools.partial(
    pl.kernel,
    out_type=jax.ShapeDtypeStruct((NC, NP, D), jnp.float32),
    mesh=_mesh,
    scratch_types=[
        pltpu.VMEM((CH, K), jnp.int32),
        pltpu.VMEM((CH, K), jnp.int32),
        pltpu.VMEM((K, D), jnp.float32),
        pltpu.VMEM((K, D), jnp.float32),
        pltpu.VMEM_SHARED((NP, D), jnp.float32),
        pltpu.SemaphoreType.DMA,
        pltpu.SemaphoreType.DMA,
    ],
)
def _aggregate_kernel(y_hbm, row_hbm, col_hbm, zeros2_hbm,
                      out_hbm, row_v, col_v, buf0, buf1, acc, sem0, sem1):
    cid = lax.axis_index("c")
    sid = lax.axis_index("s")
    wid = sid * NC + cid
    pltpu.sync_copy(zeros2_hbm, acc.at[pl.ds(sid * RPT, RPT)])
    plsc.subcore_barrier()

    def chunk(c, _):
        pltpu.sync_copy(row_hbm.at[wid, c], row_v)
        pltpu.sync_copy(col_hbm.at[wid, c], col_v)

        def body(j, _):
            b0 = 2 * j
            c0 = pltpu.async_copy(y_hbm.at[row_v.at[b0]], buf0, sem0)
            c1 = pltpu.async_copy(y_hbm.at[row_v.at[b0 + 1]], buf1, sem1)
            c0.wait()
            pltpu.sync_copy(buf0, acc.at[col_v.at[b0]], add=True)
            c1.wait()
            pltpu.sync_copy(buf1, acc.at[col_v.at[b0 + 1]], add=True)
            return ()

        lax.fori_loop(0, CH // 2, body, ())
        return ()

    lax.fori_loop(0, NCH, chunk, ())
    plsc.subcore_barrier()
    pltpu.sync_copy(acc.at[pl.ds(sid * RPT, RPT)],
                    out_hbm.at[cid, pl.ds(sid * RPT, RPT)])


def _scale_matmul_body(d0_r, d1_r, x_r, w_r, y_r, dn_r):
    d = d0_r[...] + d1_r[...]
    dn = jnp.where(d > 0.0, lax.rsqrt(d), 0.0)
    xs = x_r[...] * dn[:, None]
    y_r[...] = lax.dot_general(xs, w_r[...], (((1,), (1,)), ((), ())),
                               preferred_element_type=jnp.float32)
    dn_r[...] = dn


BN = 512
_scale_matmul = pl.pallas_call(
    _scale_matmul_body,
    grid=(NP // BN,),
    in_specs=[
        pl.BlockSpec((BN,), lambda i: (i,)),
        pl.BlockSpec((BN,), lambda i: (i,)),
        pl.BlockSpec((BN, D), lambda i: (i, 0)),
        pl.BlockSpec((D, D), lambda i: (0, 0)),
    ],
    out_specs=[
        pl.BlockSpec((BN, D), lambda i: (i, 0)),
        pl.BlockSpec((BN,), lambda i: (i,)),
    ],
    out_shape=[
        jax.ShapeDtypeStruct((NP, D), jnp.float32),
        jax.ShapeDtypeStruct((NP,), jnp.float32),
    ],
)


def _finish_body(dn_r, p0_r, p1_r, b_r, out_r):
    agg = p0_r[...] + p1_r[...]
    out_r[...] = dn_r[...][:, None] * agg + b_r[...][None, :]


_finish = pl.pallas_call(
    _finish_body,
    grid=(NP // BN,),
    in_specs=[
        pl.BlockSpec((BN,), lambda i: (i,)),
        pl.BlockSpec((BN, D), lambda i: (i, 0)),
        pl.BlockSpec((BN, D), lambda i: (i, 0)),
        pl.BlockSpec((D,), lambda i: (0,)),
    ],
    out_specs=pl.BlockSpec((BN, D), lambda i: (i, 0)),
    out_shape=jax.ShapeDtypeStruct((NP, D), jnp.float32),
)


@jax.jit
def kernel(x, edge_index, x0, W, b):
    row = edge_index[0]
    col = edge_index[1]
    pad = EP - E
    row_p = jnp.concatenate([row, jnp.zeros((pad,), jnp.int32)])
    col_p = jnp.concatenate([col, jnp.full((pad,), DUMP, jnp.int32)])
    col3w = col_p.reshape(NW, NB_DEG, K)
    row4 = row_p.reshape(NW, NCH, CH, K)
    col4 = col_p.reshape(NW, NCH, CH, K)
    x_p = jnp.pad(x, ((0, NP - N), (0, 0)))

    zeros1 = jnp.zeros((RPT,), jnp.float32)
    zeros2 = jnp.zeros((RPT, D), jnp.float32)
    ones = jnp.ones((K,), jnp.float32)

    d_part = _degree_kernel(col3w, zeros1, ones)
    y, dn = _scale_matmul(d_part[0], d_part[1], x_p, W)
    parts = _aggregate_kernel(y, row4, col4, zeros2)
    out = _finish(dn, parts[0], parts[1], b)
    return out[:N]
